# baseline (speedup 1.0000x reference)
import functools

import jax
import jax.numpy as jnp
from jax import lax
from jax.experimental import pallas as pl
from jax.experimental.pallas import tpu as pltpu
from jax.experimental.pallas import tpu_sc as plsc

N_NODES = 10000
PAD_N = 10240
IN_DIM = 128
D = 64
NC = 2
NS = 16
NW = NC * NS
CHUNK = 128
N_EDGE = 320000
CHUNK_ROWS = N_EDGE // CHUNK
CBASE = CHUNK_ROWS // NW
NIDX = CBASE + 2
NPT = PAD_N // NS
BLK = 1024

_mesh = plsc.VectorSubcoreMesh(core_axis_name="c", subcore_axis_name="s")
_sc_params = pltpu.CompilerParams(use_tc_tiling_on_sc=False)


@functools.partial(
    pl.kernel,
    out_type=jax.ShapeDtypeStruct((NC, PAD_N, 16), jnp.float32),
    mesh=_mesh,
    compiler_params=_sc_params,
    scratch_types=[
        pltpu.VMEM((NIDX, CHUNK), jnp.int32),
        pltpu.VMEM((CHUNK, 16), jnp.float32),
        pltpu.VMEM_SHARED((PAD_N, 16), jnp.float32),
    ],
)
def _deg_kernel(dst_hbm, zeros_hbm, ones_hbm, out_hbm, dst_v, ones_v, acc):
    cid = lax.axis_index("c")
    sid = lax.axis_index("s")
    wid = cid * NS + sid
    nbase = sid * NPT
    ncha = CBASE + 2 * (wid >= NW - 2)
    ebase = CBASE * wid + 2 * jnp.maximum(0, wid - (NW - 2))
    pltpu.sync_copy(zeros_hbm.at[pl.ds(nbase, NPT)], acc.at[pl.ds(nbase, NPT)])
    pltpu.sync_copy(ones_hbm, ones_v)
    pltpu.sync_copy(dst_hbm.at[pl.ds(ebase, NIDX)], dst_v)
    plsc.subcore_barrier()

    @pl.loop(0, ncha)
    def _(j):
        pltpu.sync_copy(ones_v, acc.at[dst_v.at[j]], add=True)

    plsc.subcore_barrier()
    pltpu.sync_copy(acc.at[pl.ds(nbase, NPT)],
                    out_hbm.at[cid, pl.ds(nbase, NPT)])


@functools.partial(
    pl.kernel,
    out_type=jax.ShapeDtypeStruct((NC, PAD_N, D), jnp.float32),
    mesh=_mesh,
    compiler_params=_sc_params,
    scratch_types=[
        pltpu.VMEM((NIDX, CHUNK), jnp.int32),
        pltpu.VMEM((NIDX, CHUNK), jnp.int32),
        pltpu.VMEM((CHUNK, D), jnp.float32),
        pltpu.VMEM((CHUNK, D), jnp.float32),
        pltpu.SemaphoreType.DMA,
        pltpu.SemaphoreType.DMA,
        pltpu.SemaphoreType.DMA,
        pltpu.SemaphoreType.DMA,
        pltpu.VMEM_SHARED((PAD_N, D), jnp.float32),
        pltpu.VMEM_SHARED((PAD_N, D), jnp.float32),
    ],
)
def _hop_kernel(t_hbm, src_hbm, dst_hbm, out_hbm,
                src_v, dst_v, rows0, rows1, gsem0, gsem1, ssem0, ssem1,
                acc, t_sh):
    cid = lax.axis_index("c")
    sid = lax.axis_index("s")
    wid = cid * NS + sid
    nbase = sid * NPT
    ncha = CBASE + 2 * (wid >= NW - 2)
    ebase = CBASE * wid + 2 * jnp.maximum(0, wid - (NW - 2))
    pltpu.sync_copy(t_hbm.at[pl.ds(nbase, NPT)], acc.at[pl.ds(nbase, NPT)])
    pltpu.sync_copy(t_hbm.at[pl.ds(nbase, NPT)], t_sh.at[pl.ds(nbase, NPT)])
    pltpu.sync_copy(src_hbm.at[pl.ds(ebase, NIDX)], src_v)
    pltpu.sync_copy(dst_hbm.at[pl.ds(ebase, NIDX)], dst_v)
    plsc.subcore_barrier()

    bufs = (rows0, rows1)
    gsems = (gsem0, gsem1)
    ssems = (ssem0, ssem1)
    pltpu.async_copy(t_sh.at[src_v.at[0]], rows0, gsem0)

    @pl.loop(0, ncha, step=2)
    def _(j):
        for b in range(2):
            jj = j + b
            ob = 1 - b
            pltpu.make_async_copy(t_sh.at[src_v.at[jj]], bufs[b], gsems[b]).wait()

            @pl.when(jj > 0)
            def _():
                pltpu.make_async_copy(
                    bufs[ob], acc.at[dst_v.at[jj - 1]], ssems[ob]).wait()

            pltpu.async_copy(bufs[b], acc.at[dst_v.at[jj]], ssems[b], add=True)

            @pl.when(jj + 1 < ncha)
            def _():
                pltpu.async_copy(t_sh.at[src_v.at[jj + 1]], bufs[ob], gsems[ob])

    pltpu.make_async_copy(bufs[1], acc.at[dst_v.at[ncha - 1]], ssems[1]).wait()

    plsc.subcore_barrier()
    pltpu.sync_copy(acc.at[pl.ds(nbase, NPT)],
                    out_hbm.at[cid, pl.ds(nbase, NPT)])


def _deg_cols(d_ref):
    return 1.0 + d_ref[0, :, 0:1] + d_ref[1, :, 0:1]


def _prep_body(x_ref, w_ref, d_ref, t0_ref):
    y = lax.dot_general(x_ref[...], w_ref[...], (((1,), (1,)), ((), ())),
                        preferred_element_type=jnp.float32,
                        precision=lax.Precision.HIGHEST)
    t0_ref[...] = y * lax.rsqrt(_deg_cols(d_ref))


_prep = pl.pallas_call(
    _prep_body,
    grid=(PAD_N // BLK,),
    in_specs=[
        pl.BlockSpec((BLK, IN_DIM), lambda i: (i, 0)),
        pl.BlockSpec((D, IN_DIM), lambda i: (0, 0)),
        pl.BlockSpec((NC, BLK, 16), lambda i: (0, i, 0)),
    ],
    out_specs=pl.BlockSpec((BLK, D), lambda i: (i, 0)),
    out_shape=jax.ShapeDtypeStruct((PAD_N, D), jnp.float32),
)


def _mid_body(p_ref, t_ref, d_ref, o_ref):
    o_ref[...] = (p_ref[0] + p_ref[1] - t_ref[...]) / _deg_cols(d_ref)


_mid = pl.pallas_call(
    _mid_body,
    grid=(PAD_N // BLK,),
    in_specs=[
        pl.BlockSpec((NC, BLK, D), lambda i: (0, i, 0)),
        pl.BlockSpec((BLK, D), lambda i: (i, 0)),
        pl.BlockSpec((NC, BLK, 16), lambda i: (0, i, 0)),
    ],
    out_specs=pl.BlockSpec((BLK, D), lambda i: (i, 0)),
    out_shape=jax.ShapeDtypeStruct((PAD_N, D), jnp.float32),
)


def _final_body(q_ref, t_ref, d_ref, b_ref, o_ref):
    h = (q_ref[0] + q_ref[1] - t_ref[...]) * lax.rsqrt(_deg_cols(d_ref))
    z = h + b_ref[0:1, :]
    s = z - jnp.max(z, axis=1, keepdims=True)
    o_ref[...] = s - jnp.log(jnp.sum(jnp.exp(s), axis=1, keepdims=True))


_final = pl.pallas_call(
    _final_body,
    grid=(PAD_N // BLK,),
    in_specs=[
        pl.BlockSpec((NC, BLK, D), lambda i: (0, i, 0)),
        pl.BlockSpec((BLK, D), lambda i: (i, 0)),
        pl.BlockSpec((NC, BLK, 16), lambda i: (0, i, 0)),
        pl.BlockSpec((8, D), lambda i: (0, 0)),
    ],
    out_specs=pl.BlockSpec((BLK, D), lambda i: (i, 0)),
    out_shape=jax.ShapeDtypeStruct((N_NODES, D), jnp.float32),
)


def kernel(x, edge_index, W, b):
    src2d = edge_index[0].astype(jnp.int32).reshape(CHUNK_ROWS, CHUNK)
    dst2d = edge_index[1].astype(jnp.int32).reshape(CHUNK_ROWS, CHUNK)
    zeros16 = jnp.zeros((PAD_N, 16), jnp.float32)
    ones16 = jnp.ones((CHUNK, 16), jnp.float32)
    deg16 = _deg_kernel(dst2d, zeros16, ones16)

    xp = jnp.pad(x.astype(jnp.float32), ((0, PAD_N - N_NODES), (0, 0)))
    t0 = _prep(xp, W, deg16)
    p = _hop_kernel(t0, src2d, dst2d)
    t1 = _mid(p, t0, deg16)
    q = _hop_kernel(t1, src2d, dst2d)
    b2d = jnp.broadcast_to(b.reshape(1, D), (8, D))
    return _final(q, t1, deg16, b2d)


# speedup vs baseline: 1.0060x; 1.0060x over previous
import functools

import jax
import jax.numpy as jnp
from jax import lax
from jax.experimental import pallas as pl
from jax.experimental.pallas import tpu as pltpu
from jax.experimental.pallas import tpu_sc as plsc

N_NODES = 10000
PAD_N = 10240
IN_DIM = 128
D = 64
NC = 2
NS = 16
NW = NC * NS
CHUNK = 128
N_EDGE = 320000
CHUNK_ROWS = N_EDGE // CHUNK
CBASE = CHUNK_ROWS // NW
NIDX = CBASE + 2
NPT = PAD_N // NS
BLK = 10240

_mesh = plsc.VectorSubcoreMesh(core_axis_name="c", subcore_axis_name="s")
_sc_params = pltpu.CompilerParams(use_tc_tiling_on_sc=False)


@functools.partial(
    pl.kernel,
    out_type=jax.ShapeDtypeStruct((NC, PAD_N, 16), jnp.float32),
    mesh=_mesh,
    compiler_params=_sc_params,
    scratch_types=[
        pltpu.VMEM((NIDX, CHUNK), jnp.int32),
        pltpu.VMEM((CHUNK, 16), jnp.float32),
        pltpu.VMEM_SHARED((PAD_N, 16), jnp.float32),
    ],
)
def _deg_kernel(dst_hbm, zeros_hbm, ones_hbm, out_hbm, dst_v, ones_v, acc):
    cid = lax.axis_index("c")
    sid = lax.axis_index("s")
    wid = cid * NS + sid
    nbase = sid * NPT
    ncha = CBASE + 2 * (wid >= NW - 2)
    ebase = CBASE * wid + 2 * jnp.maximum(0, wid - (NW - 2))
    pltpu.sync_copy(zeros_hbm.at[pl.ds(nbase, NPT)], acc.at[pl.ds(nbase, NPT)])
    pltpu.sync_copy(ones_hbm, ones_v)
    pltpu.sync_copy(dst_hbm.at[pl.ds(ebase, NIDX)], dst_v)
    plsc.subcore_barrier()

    @pl.loop(0, ncha)
    def _(j):
        pltpu.sync_copy(ones_v, acc.at[dst_v.at[j]], add=True)

    plsc.subcore_barrier()
    pltpu.sync_copy(acc.at[pl.ds(nbase, NPT)],
                    out_hbm.at[cid, pl.ds(nbase, NPT)])


@functools.partial(
    pl.kernel,
    out_type=jax.ShapeDtypeStruct((NC, PAD_N, D), jnp.float32),
    mesh=_mesh,
    compiler_params=_sc_params,
    scratch_types=[
        pltpu.VMEM((NIDX, CHUNK), jnp.int32),
        pltpu.VMEM((NIDX, CHUNK), jnp.int32),
        pltpu.VMEM((CHUNK, D), jnp.float32),
        pltpu.VMEM((CHUNK, D), jnp.float32),
        pltpu.SemaphoreType.DMA,
        pltpu.SemaphoreType.DMA,
        pltpu.SemaphoreType.DMA,
        pltpu.SemaphoreType.DMA,
        pltpu.VMEM_SHARED((PAD_N, D), jnp.float32),
        pltpu.VMEM_SHARED((PAD_N, D), jnp.float32),
    ],
)
def _hop_kernel(t_hbm, src_hbm, dst_hbm, out_hbm,
                src_v, dst_v, rows0, rows1, gsem0, gsem1, ssem0, ssem1,
                acc, t_sh):
    cid = lax.axis_index("c")
    sid = lax.axis_index("s")
    wid = cid * NS + sid
    nbase = sid * NPT
    ncha = CBASE + 2 * (wid >= NW - 2)
    ebase = CBASE * wid + 2 * jnp.maximum(0, wid - (NW - 2))
    pltpu.sync_copy(t_hbm.at[pl.ds(nbase, NPT)], acc.at[pl.ds(nbase, NPT)])
    pltpu.sync_copy(t_hbm.at[pl.ds(nbase, NPT)], t_sh.at[pl.ds(nbase, NPT)])
    pltpu.sync_copy(src_hbm.at[pl.ds(ebase, NIDX)], src_v)
    pltpu.sync_copy(dst_hbm.at[pl.ds(ebase, NIDX)], dst_v)
    plsc.subcore_barrier()

    bufs = (rows0, rows1)
    gsems = (gsem0, gsem1)
    ssems = (ssem0, ssem1)
    pltpu.async_copy(t_sh.at[src_v.at[0]], rows0, gsem0)

    @pl.loop(0, ncha, step=2)
    def _(j):
        for b in range(2):
            jj = j + b
            ob = 1 - b
            pltpu.make_async_copy(t_sh.at[src_v.at[jj]], bufs[b], gsems[b]).wait()

            @pl.when(jj > 0)
            def _():
                pltpu.make_async_copy(
                    bufs[ob], acc.at[dst_v.at[jj - 1]], ssems[ob]).wait()

            pltpu.async_copy(bufs[b], acc.at[dst_v.at[jj]], ssems[b], add=True)

            @pl.when(jj + 1 < ncha)
            def _():
                pltpu.async_copy(t_sh.at[src_v.at[jj + 1]], bufs[ob], gsems[ob])

    pltpu.make_async_copy(bufs[1], acc.at[dst_v.at[ncha - 1]], ssems[1]).wait()

    plsc.subcore_barrier()
    pltpu.sync_copy(acc.at[pl.ds(nbase, NPT)],
                    out_hbm.at[cid, pl.ds(nbase, NPT)])


def _deg_cols(d_ref):
    return 1.0 + d_ref[0, :, 0:1] + d_ref[1, :, 0:1]


def _prep_body(x_ref, w_ref, d_ref, t0_ref):
    y = lax.dot_general(x_ref[...], w_ref[...], (((1,), (1,)), ((), ())),
                        preferred_element_type=jnp.float32,
                        precision=lax.Precision.HIGHEST)
    t0_ref[...] = y * lax.rsqrt(_deg_cols(d_ref))


_prep = pl.pallas_call(
    _prep_body,
    grid=(PAD_N // BLK,),
    in_specs=[
        pl.BlockSpec((BLK, IN_DIM), lambda i: (i, 0)),
        pl.BlockSpec((D, IN_DIM), lambda i: (0, 0)),
        pl.BlockSpec((NC, BLK, 16), lambda i: (0, i, 0)),
    ],
    out_specs=pl.BlockSpec((BLK, D), lambda i: (i, 0)),
    out_shape=jax.ShapeDtypeStruct((PAD_N, D), jnp.float32),
)


def _mid_body(p_ref, t_ref, d_ref, o_ref):
    o_ref[...] = (p_ref[0] + p_ref[1] - t_ref[...]) / _deg_cols(d_ref)


_mid = pl.pallas_call(
    _mid_body,
    grid=(PAD_N // BLK,),
    in_specs=[
        pl.BlockSpec((NC, BLK, D), lambda i: (0, i, 0)),
        pl.BlockSpec((BLK, D), lambda i: (i, 0)),
        pl.BlockSpec((NC, BLK, 16), lambda i: (0, i, 0)),
    ],
    out_specs=pl.BlockSpec((BLK, D), lambda i: (i, 0)),
    out_shape=jax.ShapeDtypeStruct((PAD_N, D), jnp.float32),
)


def _final_body(q_ref, t_ref, d_ref, b_ref, o_ref):
    h = (q_ref[0] + q_ref[1] - t_ref[...]) * lax.rsqrt(_deg_cols(d_ref))
    z = h + b_ref[0:1, :]
    s = z - jnp.max(z, axis=1, keepdims=True)
    out = s - jnp.log(jnp.sum(jnp.exp(s), axis=1, keepdims=True))
    o_ref[...] = out[:N_NODES, :]


_final = pl.pallas_call(
    _final_body,
    grid=(1,),
    in_specs=[
        pl.BlockSpec((NC, BLK, D), lambda i: (0, 0, 0)),
        pl.BlockSpec((BLK, D), lambda i: (0, 0)),
        pl.BlockSpec((NC, BLK, 16), lambda i: (0, 0, 0)),
        pl.BlockSpec((8, D), lambda i: (0, 0)),
    ],
    out_specs=pl.BlockSpec((N_NODES, D), lambda i: (0, 0)),
    out_shape=jax.ShapeDtypeStruct((N_NODES, D), jnp.float32),
)


def kernel(x, edge_index, W, b):
    src2d = edge_index[0].astype(jnp.int32).reshape(CHUNK_ROWS, CHUNK)
    dst2d = edge_index[1].astype(jnp.int32).reshape(CHUNK_ROWS, CHUNK)
    zeros16 = jnp.zeros((PAD_N, 16), jnp.float32)
    ones16 = jnp.ones((CHUNK, 16), jnp.float32)
    deg16 = _deg_kernel(dst2d, zeros16, ones16)

    xp = jnp.pad(x.astype(jnp.float32), ((0, PAD_N - N_NODES), (0, 0)))
    t0 = _prep(xp, W, deg16)
    p = _hop_kernel(t0, src2d, dst2d)
    t1 = _mid(p, t0, deg16)
    q = _hop_kernel(t1, src2d, dst2d)
    b2d = jnp.broadcast_to(b.reshape(1, D), (8, D))
    return _final(q, t1, deg16, b2d)


# speedup vs baseline: 1.3310x; 1.3231x over previous
import functools

import jax
import jax.numpy as jnp
from jax import lax
from jax.experimental import pallas as pl
from jax.experimental.pallas import tpu as pltpu
from jax.experimental.pallas import tpu_sc as plsc

N_NODES = 10000
PAD_N = 10240
IN_DIM = 128
D = 64
NC = 2
NS = 16
NW = NC * NS
CHUNK = 128
N_EDGE = 320000
CHUNK_ROWS = N_EDGE // CHUNK
CBASE = CHUNK_ROWS // NW
NIDX = CBASE + 2
NPT = PAD_N // NS
BLK = 10240

_mesh = plsc.VectorSubcoreMesh(core_axis_name="c", subcore_axis_name="s")
_sc_params = pltpu.CompilerParams(use_tc_tiling_on_sc=False)


@functools.partial(
    pl.kernel,
    out_type=jax.ShapeDtypeStruct((NC, PAD_N, 16), jnp.float32),
    mesh=_mesh,
    compiler_params=_sc_params,
    scratch_types=[
        pltpu.VMEM((NIDX, CHUNK), jnp.int32),
        pltpu.VMEM((CHUNK, 16), jnp.float32),
        pltpu.VMEM_SHARED((PAD_N, 16), jnp.float32),
    ],
)
def _deg_kernel(dst_hbm, zeros_hbm, ones_hbm, out_hbm, dst_v, ones_v, acc):
    cid = lax.axis_index("c")
    sid = lax.axis_index("s")
    wid = cid * NS + sid
    nbase = sid * NPT
    ncha = CBASE + 2 * (wid >= NW - 2)
    ebase = CBASE * wid + 2 * jnp.maximum(0, wid - (NW - 2))
    pltpu.sync_copy(zeros_hbm.at[pl.ds(nbase, NPT)], acc.at[pl.ds(nbase, NPT)])
    pltpu.sync_copy(ones_hbm, ones_v)
    pltpu.sync_copy(dst_hbm.at[pl.ds(ebase, NIDX)], dst_v)
    plsc.subcore_barrier()

    @pl.loop(0, ncha)
    def _(j):
        pltpu.sync_copy(ones_v, acc.at[dst_v.at[j]], add=True)

    plsc.subcore_barrier()
    pltpu.sync_copy(acc.at[pl.ds(nbase, NPT)],
                    out_hbm.at[cid, pl.ds(nbase, NPT)])


@functools.partial(
    pl.kernel,
    out_type=jax.ShapeDtypeStruct((NC, PAD_N, D), jnp.bfloat16),
    mesh=_mesh,
    compiler_params=_sc_params,
    scratch_types=[
        pltpu.VMEM((NIDX, CHUNK), jnp.int32),
        pltpu.VMEM((NIDX, CHUNK), jnp.int32),
        pltpu.VMEM((CHUNK, D), jnp.bfloat16),
        pltpu.VMEM((CHUNK, D), jnp.bfloat16),
        pltpu.SemaphoreType.DMA,
        pltpu.SemaphoreType.DMA,
        pltpu.SemaphoreType.DMA,
        pltpu.SemaphoreType.DMA,
        pltpu.VMEM_SHARED((PAD_N, D), jnp.bfloat16),
        pltpu.VMEM_SHARED((PAD_N, D), jnp.bfloat16),
    ],
)
def _hop_kernel(t_hbm, src_hbm, dst_hbm, out_hbm,
                src_v, dst_v, rows0, rows1, gsem0, gsem1, ssem0, ssem1,
                acc, t_sh):
    cid = lax.axis_index("c")
    sid = lax.axis_index("s")
    wid = cid * NS + sid
    nbase = sid * NPT
    ncha = CBASE + 2 * (wid >= NW - 2)
    ebase = CBASE * wid + 2 * jnp.maximum(0, wid - (NW - 2))
    pltpu.sync_copy(t_hbm.at[pl.ds(nbase, NPT)], acc.at[pl.ds(nbase, NPT)])
    pltpu.sync_copy(t_hbm.at[pl.ds(nbase, NPT)], t_sh.at[pl.ds(nbase, NPT)])
    pltpu.sync_copy(src_hbm.at[pl.ds(ebase, NIDX)], src_v)
    pltpu.sync_copy(dst_hbm.at[pl.ds(ebase, NIDX)], dst_v)
    plsc.subcore_barrier()

    bufs = (rows0, rows1)
    gsems = (gsem0, gsem1)
    ssems = (ssem0, ssem1)
    pltpu.async_copy(t_sh.at[src_v.at[0]], rows0, gsem0)

    @pl.loop(0, ncha, step=2)
    def _(j):
        for b in range(2):
            jj = j + b
            ob = 1 - b
            pltpu.make_async_copy(t_sh.at[src_v.at[jj]], bufs[b], gsems[b]).wait()

            @pl.when(jj > 0)
            def _():
                pltpu.make_async_copy(
                    bufs[ob], acc.at[dst_v.at[jj - 1]], ssems[ob]).wait()

            pltpu.async_copy(bufs[b], acc.at[dst_v.at[jj]], ssems[b], add=True)

            @pl.when(jj + 1 < ncha)
            def _():
                pltpu.async_copy(t_sh.at[src_v.at[jj + 1]], bufs[ob], gsems[ob])

    pltpu.make_async_copy(bufs[1], acc.at[dst_v.at[ncha - 1]], ssems[1]).wait()

    plsc.subcore_barrier()
    pltpu.sync_copy(acc.at[pl.ds(nbase, NPT)],
                    out_hbm.at[cid, pl.ds(nbase, NPT)])


def _deg_cols(d_ref):
    return 1.0 + d_ref[0, :, 0:1] + d_ref[1, :, 0:1]


def _prep_body(x_ref, w_ref, d_ref, t0_ref):
    y = lax.dot_general(x_ref[...], w_ref[...], (((1,), (1,)), ((), ())),
                        preferred_element_type=jnp.float32,
                        precision=lax.Precision.HIGHEST)
    t0_ref[...] = (y * lax.rsqrt(_deg_cols(d_ref))).astype(jnp.bfloat16)


_prep = pl.pallas_call(
    _prep_body,
    grid=(PAD_N // BLK,),
    in_specs=[
        pl.BlockSpec((BLK, IN_DIM), lambda i: (i, 0)),
        pl.BlockSpec((D, IN_DIM), lambda i: (0, 0)),
        pl.BlockSpec((NC, BLK, 16), lambda i: (0, i, 0)),
    ],
    out_specs=pl.BlockSpec((BLK, D), lambda i: (i, 0)),
    out_shape=jax.ShapeDtypeStruct((PAD_N, D), jnp.bfloat16),
)


def _mid_body(p_ref, t_ref, d_ref, o_ref):
    u = (p_ref[0].astype(jnp.float32) + p_ref[1].astype(jnp.float32)
         - t_ref[...].astype(jnp.float32))
    o_ref[...] = (u / _deg_cols(d_ref)).astype(jnp.bfloat16)


_mid = pl.pallas_call(
    _mid_body,
    grid=(PAD_N // BLK,),
    in_specs=[
        pl.BlockSpec((NC, BLK, D), lambda i: (0, i, 0)),
        pl.BlockSpec((BLK, D), lambda i: (i, 0)),
        pl.BlockSpec((NC, BLK, 16), lambda i: (0, i, 0)),
    ],
    out_specs=pl.BlockSpec((BLK, D), lambda i: (i, 0)),
    out_shape=jax.ShapeDtypeStruct((PAD_N, D), jnp.bfloat16),
)


def _final_body(q_ref, t_ref, d_ref, b_ref, o_ref):
    h = (q_ref[0].astype(jnp.float32) + q_ref[1].astype(jnp.float32)
         - t_ref[...].astype(jnp.float32)) * lax.rsqrt(_deg_cols(d_ref))
    z = h + b_ref[0:1, :]
    s = z - jnp.max(z, axis=1, keepdims=True)
    out = s - jnp.log(jnp.sum(jnp.exp(s), axis=1, keepdims=True))
    o_ref[...] = out[:N_NODES, :]


_final = pl.pallas_call(
    _final_body,
    grid=(1,),
    in_specs=[
        pl.BlockSpec((NC, BLK, D), lambda i: (0, 0, 0)),
        pl.BlockSpec((BLK, D), lambda i: (0, 0)),
        pl.BlockSpec((NC, BLK, 16), lambda i: (0, 0, 0)),
        pl.BlockSpec((8, D), lambda i: (0, 0)),
    ],
    out_specs=pl.BlockSpec((N_NODES, D), lambda i: (0, 0)),
    out_shape=jax.ShapeDtypeStruct((N_NODES, D), jnp.float32),
)


def kernel(x, edge_index, W, b):
    src2d = edge_index[0].astype(jnp.int32).reshape(CHUNK_ROWS, CHUNK)
    dst2d = edge_index[1].astype(jnp.int32).reshape(CHUNK_ROWS, CHUNK)
    zeros16 = jnp.zeros((PAD_N, 16), jnp.float32)
    ones16 = jnp.ones((CHUNK, 16), jnp.float32)
    deg16 = _deg_kernel(dst2d, zeros16, ones16)

    xp = jnp.pad(x.astype(jnp.float32), ((0, PAD_N - N_NODES), (0, 0)))
    t0 = _prep(xp, W, deg16)
    p = _hop_kernel(t0, src2d, dst2d)
    t1 = _mid(p, t0, deg16)
    q = _hop_kernel(t1, src2d, dst2d)
    b2d = jnp.broadcast_to(b.reshape(1, D), (8, D))
    return _final(q, t1, deg16, b2d)


# speedup vs baseline: 1.3329x; 1.0014x over previous
import functools

import jax
import jax.numpy as jnp
from jax import lax
from jax.experimental import pallas as pl
from jax.experimental.pallas import tpu as pltpu
from jax.experimental.pallas import tpu_sc as plsc

N_NODES = 10000
PAD_N = 10240
IN_DIM = 128
D = 64
NC = 2
NS = 16
NW = NC * NS
CHUNK = 128
N_EDGE = 320000
CHUNK_ROWS = N_EDGE // CHUNK
CBASE = CHUNK_ROWS // NW
NIDX = CBASE + 2
NPT = PAD_N // NS
BLK = 10240

_mesh = plsc.VectorSubcoreMesh(core_axis_name="c", subcore_axis_name="s")
_sc_params = pltpu.CompilerParams(use_tc_tiling_on_sc=False)


@functools.partial(
    pl.kernel,
    out_type=jax.ShapeDtypeStruct((NC, PAD_N, 16), jnp.float32),
    mesh=_mesh,
    compiler_params=_sc_params,
    scratch_types=[
        pltpu.VMEM((NIDX * CHUNK,), jnp.int32),
        pltpu.VMEM((CHUNK, 16), jnp.float32),
        pltpu.VMEM_SHARED((PAD_N, 16), jnp.float32),
    ],
)
def _deg_kernel(dst_hbm, zeros_hbm, ones_hbm, out_hbm, dst_v, ones_v, acc):
    cid = lax.axis_index("c")
    sid = lax.axis_index("s")
    wid = cid * NS + sid
    nbase = sid * NPT
    ncha = CBASE + 2 * (wid >= NW - 2)
    ebase = CBASE * wid + 2 * jnp.maximum(0, wid - (NW - 2))
    pltpu.sync_copy(zeros_hbm.at[pl.ds(nbase, NPT)], acc.at[pl.ds(nbase, NPT)])
    pltpu.sync_copy(ones_hbm, ones_v)
    pltpu.sync_copy(dst_hbm.at[pl.ds(ebase * CHUNK, NIDX * CHUNK)], dst_v)
    plsc.subcore_barrier()

    @pl.loop(0, ncha)
    def _(j):
        pltpu.sync_copy(ones_v, acc.at[dst_v.at[pl.ds(j * CHUNK, CHUNK)]],
                        add=True)

    plsc.subcore_barrier()
    pltpu.sync_copy(acc.at[pl.ds(nbase, NPT)],
                    out_hbm.at[cid, pl.ds(nbase, NPT)])


@functools.partial(
    pl.kernel,
    out_type=jax.ShapeDtypeStruct((NC, PAD_N, D), jnp.bfloat16),
    mesh=_mesh,
    compiler_params=_sc_params,
    scratch_types=[
        pltpu.VMEM((NIDX * CHUNK,), jnp.int32),
        pltpu.VMEM((NIDX * CHUNK,), jnp.int32),
        pltpu.VMEM((CHUNK, D), jnp.bfloat16),
        pltpu.VMEM((CHUNK, D), jnp.bfloat16),
        pltpu.SemaphoreType.DMA,
        pltpu.SemaphoreType.DMA,
        pltpu.SemaphoreType.DMA,
        pltpu.SemaphoreType.DMA,
        pltpu.VMEM_SHARED((PAD_N, D), jnp.bfloat16),
        pltpu.VMEM_SHARED((PAD_N, D), jnp.bfloat16),
    ],
)
def _hop_kernel(t_hbm, src_hbm, dst_hbm, out_hbm,
                src_v, dst_v, rows0, rows1, gsem0, gsem1, ssem0, ssem1,
                acc, t_sh):
    cid = lax.axis_index("c")
    sid = lax.axis_index("s")
    wid = cid * NS + sid
    nbase = sid * NPT
    ncha = CBASE + 2 * (wid >= NW - 2)
    ebase = CBASE * wid + 2 * jnp.maximum(0, wid - (NW - 2))
    pltpu.sync_copy(t_hbm.at[pl.ds(nbase, NPT)], acc.at[pl.ds(nbase, NPT)])
    pltpu.sync_copy(t_hbm.at[pl.ds(nbase, NPT)], t_sh.at[pl.ds(nbase, NPT)])
    pltpu.sync_copy(src_hbm.at[pl.ds(ebase * CHUNK, NIDX * CHUNK)], src_v)
    pltpu.sync_copy(dst_hbm.at[pl.ds(ebase * CHUNK, NIDX * CHUNK)], dst_v)
    plsc.subcore_barrier()

    bufs = (rows0, rows1)
    gsems = (gsem0, gsem1)
    ssems = (ssem0, ssem1)
    pltpu.async_copy(t_sh.at[src_v.at[pl.ds(0, CHUNK)]], rows0, gsem0)

    @pl.loop(0, ncha, step=2)
    def _(j):
        for b in range(2):
            jj = j + b
            ob = 1 - b
            pltpu.make_async_copy(t_sh.at[src_v.at[pl.ds(jj * CHUNK, CHUNK)]],
                                  bufs[b], gsems[b]).wait()

            @pl.when(jj > 0)
            def _():
                pltpu.make_async_copy(
                    bufs[ob], acc.at[dst_v.at[pl.ds((jj - 1) * CHUNK, CHUNK)]],
                    ssems[ob]).wait()

            pltpu.async_copy(bufs[b], acc.at[dst_v.at[pl.ds(jj * CHUNK, CHUNK)]],
                             ssems[b], add=True)

            @pl.when(jj + 1 < ncha)
            def _():
                pltpu.async_copy(t_sh.at[src_v.at[pl.ds((jj + 1) * CHUNK, CHUNK)]],
                                 bufs[ob], gsems[ob])

    pltpu.make_async_copy(
        bufs[1], acc.at[dst_v.at[pl.ds((ncha - 1) * CHUNK, CHUNK)]],
        ssems[1]).wait()

    plsc.subcore_barrier()
    pltpu.sync_copy(acc.at[pl.ds(nbase, NPT)],
                    out_hbm.at[cid, pl.ds(nbase, NPT)])


def _deg_cols(d_ref):
    return 1.0 + d_ref[0, :, 0:1] + d_ref[1, :, 0:1]


def _prep_body(x_ref, w_ref, d_ref, t0_ref):
    y = lax.dot_general(x_ref[...], w_ref[...], (((1,), (1,)), ((), ())),
                        preferred_element_type=jnp.float32,
                        precision=lax.Precision.HIGHEST)
    t0_ref[...] = (y * lax.rsqrt(_deg_cols(d_ref))).astype(jnp.bfloat16)


_prep = pl.pallas_call(
    _prep_body,
    grid=(PAD_N // BLK,),
    in_specs=[
        pl.BlockSpec((BLK, IN_DIM), lambda i: (i, 0)),
        pl.BlockSpec((D, IN_DIM), lambda i: (0, 0)),
        pl.BlockSpec((NC, BLK, 16), lambda i: (0, i, 0)),
    ],
    out_specs=pl.BlockSpec((BLK, D), lambda i: (i, 0)),
    out_shape=jax.ShapeDtypeStruct((PAD_N, D), jnp.bfloat16),
)


def _mid_body(p_ref, t_ref, d_ref, o_ref):
    u = (p_ref[0].astype(jnp.float32) + p_ref[1].astype(jnp.float32)
         - t_ref[...].astype(jnp.float32))
    o_ref[...] = (u / _deg_cols(d_ref)).astype(jnp.bfloat16)


_mid = pl.pallas_call(
    _mid_body,
    grid=(PAD_N // BLK,),
    in_specs=[
        pl.BlockSpec((NC, BLK, D), lambda i: (0, i, 0)),
        pl.BlockSpec((BLK, D), lambda i: (i, 0)),
        pl.BlockSpec((NC, BLK, 16), lambda i: (0, i, 0)),
    ],
    out_specs=pl.BlockSpec((BLK, D), lambda i: (i, 0)),
    out_shape=jax.ShapeDtypeStruct((PAD_N, D), jnp.bfloat16),
)


def _final_body(q_ref, t_ref, d_ref, b_ref, o_ref):
    h = (q_ref[0].astype(jnp.float32) + q_ref[1].astype(jnp.float32)
         - t_ref[...].astype(jnp.float32)) * lax.rsqrt(_deg_cols(d_ref))
    z = h + b_ref[0:1, :]
    s = z - jnp.max(z, axis=1, keepdims=True)
    out = s - jnp.log(jnp.sum(jnp.exp(s), axis=1, keepdims=True))
    o_ref[...] = out[:N_NODES, :]


_final = pl.pallas_call(
    _final_body,
    grid=(1,),
    in_specs=[
        pl.BlockSpec((NC, BLK, D), lambda i: (0, 0, 0)),
        pl.BlockSpec((BLK, D), lambda i: (0, 0)),
        pl.BlockSpec((NC, BLK, 16), lambda i: (0, 0, 0)),
        pl.BlockSpec((8, D), lambda i: (0, 0)),
    ],
    out_specs=pl.BlockSpec((N_NODES, D), lambda i: (0, 0)),
    out_shape=jax.ShapeDtypeStruct((N_NODES, D), jnp.float32),
)


def kernel(x, edge_index, W, b):
    src1d = edge_index[0].astype(jnp.int32)
    dst1d = edge_index[1].astype(jnp.int32)
    zeros16 = jnp.zeros((PAD_N, 16), jnp.float32)
    ones16 = jnp.ones((CHUNK, 16), jnp.float32)
    deg16 = _deg_kernel(dst1d, zeros16, ones16)

    xp = jnp.pad(x.astype(jnp.float32), ((0, PAD_N - N_NODES), (0, 0)))
    t0 = _prep(xp, W, deg16)
    p = _hop_kernel(t0, src1d, dst1d)
    t1 = _mid(p, t0, deg16)
    q = _hop_kernel(t1, src1d, dst1d)
    b2d = jnp.broadcast_to(b.reshape(1, D), (8, D))
    return _final(q, t1, deg16, b2d)
